# baseline (speedup 1.0000x reference)
import functools

import jax
import jax.numpy as jnp
from jax import lax
from jax.experimental import pallas as pl
from jax.experimental.pallas import tpu as pltpu
from jax.experimental.pallas import tpu_sc as plsc

_EPS = 1e-6
_L = 16


def _rsqrt_newton(x):
    i = plsc.bitcast(x, jnp.int32)
    i = jnp.full(x.shape, 0x5F3759DF, jnp.int32) - lax.shift_right_logical(i, 1)
    y = plsc.bitcast(i, jnp.float32)
    for _ in range(3):
        y = y * (1.5 - 0.5 * x * y * y)
    return y


@functools.lru_cache(maxsize=None)
def _build(total, M, B):
    mesh = plsc.VectorSubcoreMesh(core_axis_name="c", subcore_axis_name="s")
    NC, NS = mesh.num_cores, mesh.num_subcores
    NW = NC * NS
    SUB = 512
    RS = total // NS
    RN = total // NW
    NTS = RS // SUB
    NTN = RN // SUB
    CV = M // _L
    assert total == NS * NTS * SUB == NW * NTN * SUB and M % _L == 0

    def seg_bounds(off_vec, i, base):
        oa = off_vec[i]
        ob = jnp.int32(total) if i == B - 1 else off_vec[i + 1]
        a = jnp.clip(oa - base, 0, SUB)
        b = jnp.clip(ob - base, 0, SUB)
        return a, b

    @functools.partial(
        pl.kernel,
        out_type=(jax.ShapeDtypeStruct((total, M), jnp.float32),
                  jax.ShapeDtypeStruct((NC, NS, 2 * _L), jnp.float32)),
        mesh=mesh,
        compiler_params=pltpu.CompilerParams(needs_layout_passes=False,
                                             use_tc_tiling_on_sc=True),
        scratch_types=[
            pltpu.VMEM((SUB, M), jnp.float32),
            pltpu.VMEM((_L,), jnp.int32),
            pltpu.VMEM((2 * _L,), jnp.float32),
            pltpu.VMEM((NS, 2 * _L), jnp.float32),
        ],
    )
    def fused_k(values_hbm, offsets_hbm, out_hbm, part_hbm,
                chunk, offs, stat_v, gath_v):
        zeros = jnp.zeros((_L,), jnp.float32)
        lane_iota = lax.iota(jnp.int32, _L)
        sid = lax.axis_index("s")
        wid = lax.axis_index("c") * NS + sid
        pltpu.sync_copy(offsets_hbm.at[pl.ds(0, _L)], offs)
        off_vec = offs[...]

        sums_vec = zeros
        sq_vec = zeros
        for t in range(NTS):
            base = sid * RS + t * SUB
            pltpu.sync_copy(values_hbm.at[pl.ds(base, SUB)], chunk)
            for i in range(B):
                a, b = seg_bounds(off_vec, i, base)

                def body(r, carry):
                    s, q = carry
                    for cc in range(CV):
                        v = chunk[r, pl.ds(cc * _L, _L)]
                        s = s + v
                        q = q + v * v
                    return s, q

                s, q = lax.fori_loop(a, b, body, (zeros, zeros))
                lane = lane_iota == i
                sums_vec = jnp.where(lane, sums_vec + jnp.sum(s), sums_vec)
                sq_vec = jnp.where(lane, sq_vec + jnp.sum(q), sq_vec)

        cid = lax.axis_index("c")
        stat_v[pl.ds(0, _L)] = sums_vec
        stat_v[pl.ds(_L, _L)] = sq_vec
        pltpu.sync_copy(stat_v, part_hbm.at[cid, sid])
        plsc.subcore_barrier()
        pltpu.sync_copy(part_hbm.at[cid], gath_v)

        sums = zeros
        sqs = zeros
        for w in range(NS):
            sums = sums + gath_v[w, pl.ds(0, _L)]
            sqs = sqs + gath_v[w, pl.ds(_L, _L)]

        off_hi = jnp.full((_L,), total, jnp.int32)
        for i in range(B - 1):
            off_hi = jnp.where(lane_iota == i, off_vec[i + 1], off_hi)
        n_elem = (off_hi - off_vec).astype(jnp.float32) * jnp.float32(M)

        mean = sums / n_elem
        var = sqs / n_elem - mean * mean
        rstd = _rsqrt_newton(var + _EPS)

        for t in range(NTN):
            base = wid * RN + t * SUB
            pltpu.sync_copy(values_hbm.at[pl.ds(base, SUB)], chunk)
            for i in range(B):
                a, b = seg_bounds(off_vec, i, base)
                mv = jnp.broadcast_to(mean[i], (_L,))
                rv = jnp.broadcast_to(rstd[i], (_L,))

                def body(r):
                    for cc in range(CV):
                        v = chunk[r, pl.ds(cc * _L, _L)]
                        chunk[r, pl.ds(cc * _L, _L)] = (v - mv) * rv

                plsc.parallel_loop(a, b, unroll=4)(body)

            pltpu.sync_copy(chunk, out_hbm.at[pl.ds(base, SUB)])

    return fused_k


def kernel(values, offsets, M):
    total, m = values.shape
    B = offsets.shape[0] - 1
    out, _ = _build(total, m, B)(values, offsets)
    return out


# speedup vs baseline: 1.9443x; 1.9443x over previous
import functools

import jax
import jax.numpy as jnp
from jax import lax
from jax.experimental import pallas as pl
from jax.experimental.pallas import tpu as pltpu
from jax.experimental.pallas import tpu_sc as plsc

_EPS = 1e-6
_L = 16


def _rsqrt_newton(x):
    i = plsc.bitcast(x, jnp.int32)
    i = jnp.full(x.shape, 0x5F3759DF, jnp.int32) - lax.shift_right_logical(i, 1)
    y = plsc.bitcast(i, jnp.float32)
    for _ in range(3):
        y = y * (1.5 - 0.5 * x * y * y)
    return y


@functools.lru_cache(maxsize=None)
def _build(total, M, B):
    mesh = plsc.VectorSubcoreMesh(core_axis_name="c", subcore_axis_name="s")
    NC, NS = mesh.num_cores, mesh.num_subcores
    NW = NC * NS
    CW = total // NW
    SUB = 512
    NT = CW // SUB
    KV = SUB // _L
    assert total == NW * NT * SUB

    def seg_cols(off_vec, i):
        lo = off_vec[i]
        hi = jnp.int32(total) if i == B - 1 else off_vec[i + 1]
        return lo, hi

    @functools.partial(
        pl.kernel,
        out_type=jax.ShapeDtypeStruct((NW * 2 * _L,), jnp.float32),
        mesh=mesh,
        compiler_params=pltpu.CompilerParams(needs_layout_passes=False),
        scratch_types=[
            pltpu.VMEM((M, SUB), jnp.float32),
            pltpu.VMEM((SUB,), jnp.float32),
            pltpu.VMEM((SUB,), jnp.float32),
            pltpu.VMEM((_L,), jnp.int32),
            pltpu.VMEM((2 * _L,), jnp.float32),
        ],
    )
    def stats_k(vt_hbm, offsets_hbm, part_hbm,
                chunk, colsum, colsq, offs, stat_v):
        zeros = jnp.zeros((_L,), jnp.float32)
        lane_iota = lax.iota(jnp.int32, _L)
        wid = lax.axis_index("c") * NS + lax.axis_index("s")
        pltpu.sync_copy(offsets_hbm.at[pl.ds(0, _L)], offs)
        off_vec = offs[...]

        sums_vec = zeros
        sq_vec = zeros
        for t in range(NT):
            cbase = wid * CW + t * SUB
            pltpu.sync_copy(vt_hbm.at[:, pl.ds(cbase, SUB)], chunk)

            def kbody(k, _):
                def mbody(m, carry):
                    s, q = carry
                    v = chunk[m, pl.ds(k * _L, _L)]
                    return s + v, q + v * v

                s, q = plsc.parallel_loop(
                    0, M, unroll=8, carry=(zeros, zeros))(mbody)
                colsum[pl.ds(k * _L, _L)] = s
                colsq[pl.ds(k * _L, _L)] = q
                return 0

            lax.fori_loop(0, KV, kbody, 0)

            for i in range(B):
                lo, hi = seg_cols(off_vec, i)
                ra = jnp.clip(lo - cbase, 0, SUB)
                rb = jnp.clip(hi - cbase, 0, SUB)

                def sbody(kk, carry):
                    s, q = carry
                    g = kk * _L + lane_iota
                    msk = (g >= ra) & (g < rb)
                    s = s + jnp.where(msk, colsum[pl.ds(kk * _L, _L)], 0.0)
                    q = q + jnp.where(msk, colsq[pl.ds(kk * _L, _L)], 0.0)
                    return s, q

                s, q = lax.fori_loop(
                    lax.div(ra, _L), lax.div(rb + (_L - 1), _L),
                    sbody, (zeros, zeros))
                lane = lane_iota == i
                sums_vec = jnp.where(lane, sums_vec + jnp.sum(s), sums_vec)
                sq_vec = jnp.where(lane, sq_vec + jnp.sum(q), sq_vec)

        stat_v[pl.ds(0, _L)] = sums_vec
        stat_v[pl.ds(_L, _L)] = sq_vec
        pltpu.sync_copy(stat_v, part_hbm.at[pl.ds(wid * 2 * _L, 2 * _L)])

    @functools.partial(
        pl.kernel,
        out_type=jax.ShapeDtypeStruct((M, total), jnp.float32),
        mesh=mesh,
        compiler_params=pltpu.CompilerParams(needs_layout_passes=False),
        scratch_types=[
            pltpu.VMEM((M, SUB), jnp.float32),
            pltpu.VMEM((_L,), jnp.int32),
            pltpu.VMEM((NW * 2 * _L,), jnp.float32),
        ],
    )
    def norm_k(vt_hbm, offsets_hbm, part_hbm, out_hbm, chunk, offs, part_v):
        zeros = jnp.zeros((_L,), jnp.float32)
        lane_iota = lax.iota(jnp.int32, _L)
        wid = lax.axis_index("c") * NS + lax.axis_index("s")
        pltpu.sync_copy(offsets_hbm.at[pl.ds(0, _L)], offs)
        pltpu.sync_copy(part_hbm, part_v)
        off_vec = offs[...]

        sums = zeros
        sqs = zeros
        for w in range(NW):
            sums = sums + part_v[pl.ds(w * 2 * _L, _L)]
            sqs = sqs + part_v[pl.ds(w * 2 * _L + _L, _L)]

        off_hi = jnp.full((_L,), total, jnp.int32)
        for i in range(B - 1):
            off_hi = jnp.where(lane_iota == i, off_vec[i + 1], off_hi)
        n_elem = (off_hi - off_vec).astype(jnp.float32) * jnp.float32(M)

        mean = sums / n_elem
        var = sqs / n_elem - mean * mean
        rstd = _rsqrt_newton(var + _EPS)

        for t in range(NT):
            cbase = wid * CW + t * SUB
            pltpu.sync_copy(vt_hbm.at[:, pl.ds(cbase, SUB)], chunk)

            def kbody(k, _):
                col = cbase + k * _L + lane_iota
                seg = jnp.zeros((_L,), jnp.int32)
                for j in range(1, B):
                    seg = seg + (col >= off_vec[j]).astype(jnp.int32)
                mv = mean.at[seg].get(mode="promise_in_bounds")
                rv = rstd.at[seg].get(mode="promise_in_bounds")

                def mbody(m):
                    v = chunk[m, pl.ds(k * _L, _L)]
                    chunk[m, pl.ds(k * _L, _L)] = (v - mv) * rv

                plsc.parallel_loop(0, M, unroll=8)(mbody)
                return 0

            lax.fori_loop(0, KV, kbody, 0)
            pltpu.sync_copy(chunk, out_hbm.at[:, pl.ds(cbase, SUB)])

    return stats_k, norm_k


def kernel(values, offsets, M):
    total, m = values.shape
    B = offsets.shape[0] - 1
    stats_k, norm_k = _build(total, m, B)
    vt = values.T
    part = stats_k(vt, offsets)
    out_t = norm_k(vt, offsets, part)
    return out_t.T
